# Initial kernel scaffold; baseline (speedup 1.0000x reference)
import jax
import jax.numpy as jnp
from jax.experimental import pallas as pl


def kernel(x, edge_index, conv0_Wl, conv0_bl, conv0_Wr, conv1_Wl, conv1_bl, conv1_Wr, kan_W1, kan_b1, kan_W2, kan_b2, cls_W, cls_b):
    raise NotImplementedError("write your pallas kernel here")


# speedup vs baseline: 3.9122x; 3.9122x over previous
import jax
import jax.numpy as jnp
from jax import lax
from jax.experimental import pallas as pl
from jax.experimental.pallas import tpu as pltpu
from jax.experimental.pallas import tpu_sc as plsc

N = 10000
D = 128
HIDQ = 512
NPAD = 10240
TRASH = N
E = 320000
NW = 32
B = 64
K = -(-E // (NW * B))
EPAD = NW * B * K
RPT = NPAD // 16
DEGW = 16
W0 = D + DEGW
RB = 1280
G = NPAD // RB


def _set2d(ref, nrows, ncols, val):
    v = jnp.full((16,), val, jnp.float32)

    def body(i, carry):
        for q in range(ncols // 16):
            ref[i, pl.ds(q * 16, 16)] = v
        return carry

    lax.fori_loop(0, nrows, body, None)


_SC_MESH = plsc.VectorSubcoreMesh(core_axis_name="c", subcore_axis_name="s")


def _sc_agg_body(table, srcw, dstw, out, sidx, didx, rows, acc, sem):
    c = lax.axis_index("c")
    s = lax.axis_index("s")
    wid = c * 16 + s
    _set2d(rows, B, D, 0.0)
    for k in range(RPT // B):
        pltpu.sync_copy(rows, acc.at[pl.ds(s * RPT + k * B, B)])
    plsc.subcore_barrier()

    def step(j, carry):
        pltpu.sync_copy(srcw.at[wid, j], sidx)
        pltpu.sync_copy(dstw.at[wid, j], didx)
        pltpu.async_copy(table.at[sidx], rows, sem).wait()
        pltpu.sync_copy(rows, acc.at[didx], add=True)
        return carry

    lax.fori_loop(0, K, step, None)
    plsc.subcore_barrier()
    r0 = s * RPT
    pltpu.sync_copy(acc.at[pl.ds(r0, RPT)], out.at[c, pl.ds(r0, RPT)])


_sc_agg = pl.kernel(
    _sc_agg_body,
    out_type=jax.ShapeDtypeStruct((2, NPAD, D), jnp.float32),
    mesh=_SC_MESH,
    scratch_types=[
        pltpu.VMEM((B,), jnp.int32),
        pltpu.VMEM((B,), jnp.int32),
        pltpu.VMEM((B, D), jnp.float32),
        pltpu.VMEM_SHARED((NPAD, D), jnp.float32),
        pltpu.SemaphoreType.DMA,
    ],
)


def _sc_deg_body(dstw, out, didx, ones, dacc):
    c = lax.axis_index("c")
    s = lax.axis_index("s")
    wid = c * 16 + s
    _set2d(ones, B, D, 0.0)
    for k in range(RPT // B):
        pltpu.sync_copy(ones, dacc.at[pl.ds(s * RPT + k * B, B)])
    _set2d(ones, B, D, 1.0)
    plsc.subcore_barrier()

    def step(j, carry):
        pltpu.sync_copy(dstw.at[wid, j], didx)
        pltpu.sync_copy(ones, dacc.at[didx], add=True)
        return carry

    lax.fori_loop(0, K, step, None)
    plsc.subcore_barrier()
    r0 = s * RPT
    pltpu.sync_copy(dacc.at[pl.ds(r0, RPT)], out.at[c, pl.ds(r0, RPT)])


_sc_deg = pl.kernel(
    _sc_deg_body,
    out_type=jax.ShapeDtypeStruct((2, NPAD, D), jnp.float32),
    mesh=_SC_MESH,
    scratch_types=[
        pltpu.VMEM((B,), jnp.int32),
        pltpu.VMEM((B, D), jnp.float32),
        pltpu.VMEM_SHARED((NPAD, D), jnp.float32),
    ],
)


def _dense0_body(x_ref, wl, wr, bl, xl_o, xr_o):
    xv = x_ref[...]
    xl_o[...] = jnp.dot(xv, wl[...], preferred_element_type=jnp.float32)
    xr_o[...] = jnp.dot(xv, wr[...], preferred_element_type=jnp.float32) + bl[...]


_dense0 = pl.pallas_call(
    _dense0_body,
    grid=(G,),
    in_specs=[
        pl.BlockSpec((RB, D), lambda i: (i, 0)),
        pl.BlockSpec((D, D), lambda i: (0, 0)),
        pl.BlockSpec((D, D), lambda i: (0, 0)),
        pl.BlockSpec((1, D), lambda i: (0, 0)),
    ],
    out_specs=[pl.BlockSpec((RB, D), lambda i: (i, 0))] * 2,
    out_shape=[jax.ShapeDtypeStruct((NPAD, D), jnp.float32)] * 2,
)


def _dense1_body(p0, p1, d0, d1, xr, wl, wr, bl, h1l_o, h1r_o):
    deg = d0[:, 0:1] + d1[:, 0:1]
    invd = 1.0 / jnp.maximum(deg, 1.0)
    h1 = jnp.maximum((p0[...] + p1[...]) * invd + xr[...], 0.0)
    h1l_o[...] = jnp.dot(h1, wl[...], preferred_element_type=jnp.float32)
    h1r_o[...] = jnp.dot(h1, wr[...], preferred_element_type=jnp.float32) + bl[...]


_dense1 = pl.pallas_call(
    _dense1_body,
    grid=(G,),
    in_specs=[
        pl.BlockSpec((RB, D), lambda i: (i, 0)),
        pl.BlockSpec((RB, D), lambda i: (i, 0)),
        pl.BlockSpec((RB, D), lambda i: (i, 0)),
        pl.BlockSpec((RB, D), lambda i: (i, 0)),
        pl.BlockSpec((RB, D), lambda i: (i, 0)),
        pl.BlockSpec((D, D), lambda i: (0, 0)),
        pl.BlockSpec((D, D), lambda i: (0, 0)),
        pl.BlockSpec((1, D), lambda i: (0, 0)),
    ],
    out_specs=[pl.BlockSpec((RB, D), lambda i: (i, 0))] * 2,
    out_shape=[jax.ShapeDtypeStruct((NPAD, D), jnp.float32)] * 2,
)


def _head_body(q0, q1, d0, d1, h1r, w1, b1, w2, b2, wc, bc, out):
    deg = d0[:, 0:1] + d1[:, 0:1]
    invd = 1.0 / jnp.maximum(deg, 1.0)
    h2 = jnp.maximum((q0[...] + q1[...]) * invd + h1r[...], 0.0)
    h3 = jax.nn.gelu(jnp.dot(h2, w1[...], preferred_element_type=jnp.float32) + b1[...])
    h4 = jax.nn.gelu(jnp.dot(h3, w2[...], preferred_element_type=jnp.float32) + b2[...])
    out[...] = jnp.dot(h4, wc[...], preferred_element_type=jnp.float32) + bc[...]


_head = pl.pallas_call(
    _head_body,
    grid=(G,),
    in_specs=[
        pl.BlockSpec((RB, D), lambda i: (i, 0)),
        pl.BlockSpec((RB, D), lambda i: (i, 0)),
        pl.BlockSpec((RB, D), lambda i: (i, 0)),
        pl.BlockSpec((RB, D), lambda i: (i, 0)),
        pl.BlockSpec((RB, D), lambda i: (i, 0)),
        pl.BlockSpec((D, HIDQ), lambda i: (0, 0)),
        pl.BlockSpec((1, HIDQ), lambda i: (0, 0)),
        pl.BlockSpec((HIDQ, D), lambda i: (0, 0)),
        pl.BlockSpec((1, D), lambda i: (0, 0)),
        pl.BlockSpec((D, D), lambda i: (0, 0)),
        pl.BlockSpec((1, D), lambda i: (0, 0)),
    ],
    out_specs=pl.BlockSpec((RB, D), lambda i: (i, 0)),
    out_shape=jax.ShapeDtypeStruct((NPAD, D), jnp.float32),
)


def kernel(x, edge_index, conv0_Wl, conv0_bl, conv0_Wr,
           conv1_Wl, conv1_bl, conv1_Wr,
           kan_W1, kan_b1, kan_W2, kan_b2, cls_W, cls_b):
    src = edge_index[0].astype(jnp.int32)
    dst = edge_index[1].astype(jnp.int32)
    pad = EPAD - E
    pad_src = jnp.arange(pad, dtype=jnp.int32) % N
    pad_dst = N + jnp.arange(pad, dtype=jnp.int32) % (NPAD - N)
    srcw = jnp.concatenate([src, pad_src]).reshape(NW, K, B)
    dstw = jnp.concatenate([dst, pad_dst]).reshape(NW, K, B)
    xpad = jnp.zeros((NPAD, D), jnp.float32).at[:N].set(x)

    xl, xr = _dense0(xpad, conv0_Wl, conv0_Wr, conv0_bl.reshape(1, D))
    agg0 = _sc_agg(xl, srcw, dstw)
    degp = _sc_deg(dstw)
    p0, p1 = agg0[0], agg0[1]
    d0, d1 = degp[0], degp[1]
    h1l, h1r = _dense1(p0, p1, d0, d1, xr,
                       conv1_Wl, conv1_Wr, conv1_bl.reshape(1, D))
    agg1 = _sc_agg(h1l, srcw, dstw)
    wc = jnp.zeros((D, D), jnp.float32).at[:, 0:1].set(cls_W)
    bc = jnp.broadcast_to(cls_b.reshape(1, 1), (1, D))
    out = _head(agg1[0], agg1[1], d0, d1, h1r,
                kan_W1, kan_b1.reshape(1, HIDQ), kan_W2, kan_b2.reshape(1, D),
                wc, bc)
    return out[:N, 0]


# speedup vs baseline: 9.1229x; 2.3319x over previous
import jax
import jax.numpy as jnp
from jax import lax
from jax.experimental import pallas as pl
from jax.experimental.pallas import tpu as pltpu
from jax.experimental.pallas import tpu_sc as plsc

N = 10000
D = 128
HIDQ = 512
NPAD = 10240
TRASH = N
E = 320000
NW = 32
B = 128
K = 80
EPAD = NW * B * K
RPT = NPAD // 16
DEGW = 16
W0 = D + DEGW
RB = 1280
G = NPAD // RB


def _set2d(ref, nrows, ncols, val):
    v = jnp.full((16,), val, jnp.float32)

    def body(i, carry):
        for q in range(ncols // 16):
            ref[i, pl.ds(q * 16, 16)] = v
        return carry

    lax.fori_loop(0, nrows, body, None)


_SC_MESH = plsc.VectorSubcoreMesh(core_axis_name="c", subcore_axis_name="s")


def _sc_agg_body(table, srcw, dstw, out,
                 sidx, didx, rows0, rows1, acc,
                 isems, gsem, ssems):
    c = lax.axis_index("c")
    s = lax.axis_index("s")
    wid = c * 16 + s
    rows = (rows0, rows1)
    _set2d(rows0, B, D, 0.0)
    for k in range(RPT // B):
        pltpu.sync_copy(rows0, acc.at[pl.ds(s * RPT + k * B, B)])
    plsc.subcore_barrier()

    for u in range(2):
        pltpu.async_copy(srcw.at[wid, u], sidx.at[u], isems.at[u])
        pltpu.async_copy(dstw.at[wid, u], didx.at[u], isems.at[u])

    def outer(g, carry):
        for u in range(4):
            j = g * 4 + u
            b = u % 2
            pltpu.make_async_copy(srcw.at[wid, j], sidx.at[u], isems.at[u]).wait()
            pltpu.make_async_copy(dstw.at[wid, j], didx.at[u], isems.at[u]).wait()
            @pl.when(j >= 2)
            def _():
                pltpu.make_async_copy(table.at[pl.ds(0, B)], rows[b],
                                      ssems.at[b]).wait()
            pltpu.async_copy(table.at[sidx.at[u]], rows[b], gsem).wait()
            pltpu.async_copy(rows[b], acc.at[didx.at[u]], ssems.at[b], add=True)
            @pl.when(j + 2 < K)
            def _():
                u2 = (u + 2) % 4
                pltpu.async_copy(srcw.at[wid, j + 2], sidx.at[u2], isems.at[u2])
                pltpu.async_copy(dstw.at[wid, j + 2], didx.at[u2], isems.at[u2])
        return carry

    lax.fori_loop(0, K // 4, outer, None)
    for b in range(2):
        pltpu.make_async_copy(table.at[pl.ds(0, B)], rows[b], ssems.at[b]).wait()
    plsc.subcore_barrier()
    r0 = s * RPT
    pltpu.sync_copy(acc.at[pl.ds(r0, RPT)], out.at[c, pl.ds(r0, RPT)])


_sc_agg = pl.kernel(
    _sc_agg_body,
    out_type=jax.ShapeDtypeStruct((2, NPAD, D), jnp.float32),
    mesh=_SC_MESH,
    scratch_types=[
        pltpu.VMEM((4, B), jnp.int32),
        pltpu.VMEM((4, B), jnp.int32),
        pltpu.VMEM((B, D), jnp.float32),
        pltpu.VMEM((B, D), jnp.float32),
        pltpu.VMEM_SHARED((NPAD, D), jnp.float32),
        pltpu.SemaphoreType.DMA((4,)),
        pltpu.SemaphoreType.DMA,
        pltpu.SemaphoreType.DMA((2,)),
    ],
)


def _sc_deg_body(dstw, out, didx, ones, dacc, isems, ssems):
    c = lax.axis_index("c")
    s = lax.axis_index("s")
    wid = c * 16 + s
    _set2d(ones, B, D, 0.0)
    for k in range(RPT // B):
        pltpu.sync_copy(ones, dacc.at[pl.ds(s * RPT + k * B, B)])
    _set2d(ones, B, D, 1.0)
    plsc.subcore_barrier()

    for u in range(2):
        pltpu.async_copy(dstw.at[wid, u], didx.at[u], isems.at[u])

    def outer(g, carry):
        for u in range(4):
            j = g * 4 + u
            b = u % 2
            pltpu.make_async_copy(dstw.at[wid, j], didx.at[u], isems.at[u]).wait()
            @pl.when(j >= 2)
            def _():
                pltpu.make_async_copy(out.at[0, pl.ds(0, B)], ones,
                                      ssems.at[b]).wait()
            pltpu.async_copy(ones, dacc.at[didx.at[u]], ssems.at[b], add=True)
            @pl.when(j + 2 < K)
            def _():
                u2 = (u + 2) % 4
                pltpu.async_copy(dstw.at[wid, j + 2], didx.at[u2], isems.at[u2])
        return carry

    lax.fori_loop(0, K // 4, outer, None)
    for b in range(2):
        pltpu.make_async_copy(out.at[0, pl.ds(0, B)], ones, ssems.at[b]).wait()
    plsc.subcore_barrier()
    r0 = s * RPT
    pltpu.sync_copy(dacc.at[pl.ds(r0, RPT)], out.at[c, pl.ds(r0, RPT)])


_sc_deg = pl.kernel(
    _sc_deg_body,
    out_type=jax.ShapeDtypeStruct((2, NPAD, D), jnp.float32),
    mesh=_SC_MESH,
    scratch_types=[
        pltpu.VMEM((4, B), jnp.int32),
        pltpu.VMEM((B, D), jnp.float32),
        pltpu.VMEM_SHARED((NPAD, D), jnp.float32),
        pltpu.SemaphoreType.DMA((4,)),
        pltpu.SemaphoreType.DMA((2,)),
    ],
)


def _dense0_body(x_ref, wl, wr, bl, xl_o, xr_o):
    xv = x_ref[...]
    xl_o[...] = jnp.dot(xv, wl[...], preferred_element_type=jnp.float32)
    xr_o[...] = jnp.dot(xv, wr[...], preferred_element_type=jnp.float32) + bl[...]


_dense0 = pl.pallas_call(
    _dense0_body,
    grid=(G,),
    in_specs=[
        pl.BlockSpec((RB, D), lambda i: (i, 0)),
        pl.BlockSpec((D, D), lambda i: (0, 0)),
        pl.BlockSpec((D, D), lambda i: (0, 0)),
        pl.BlockSpec((1, D), lambda i: (0, 0)),
    ],
    out_specs=[pl.BlockSpec((RB, D), lambda i: (i, 0))] * 2,
    out_shape=[jax.ShapeDtypeStruct((NPAD, D), jnp.float32)] * 2,
)


def _dense1_body(p0, p1, d0, d1, xr, wl, wr, bl, h1l_o, h1r_o):
    deg = d0[:, 0:1] + d1[:, 0:1]
    invd = 1.0 / jnp.maximum(deg, 1.0)
    h1 = jnp.maximum((p0[...] + p1[...]) * invd + xr[...], 0.0)
    h1l_o[...] = jnp.dot(h1, wl[...], preferred_element_type=jnp.float32)
    h1r_o[...] = jnp.dot(h1, wr[...], preferred_element_type=jnp.float32) + bl[...]


_dense1 = pl.pallas_call(
    _dense1_body,
    grid=(G,),
    in_specs=[
        pl.BlockSpec((RB, D), lambda i: (i, 0)),
        pl.BlockSpec((RB, D), lambda i: (i, 0)),
        pl.BlockSpec((RB, D), lambda i: (i, 0)),
        pl.BlockSpec((RB, D), lambda i: (i, 0)),
        pl.BlockSpec((RB, D), lambda i: (i, 0)),
        pl.BlockSpec((D, D), lambda i: (0, 0)),
        pl.BlockSpec((D, D), lambda i: (0, 0)),
        pl.BlockSpec((1, D), lambda i: (0, 0)),
    ],
    out_specs=[pl.BlockSpec((RB, D), lambda i: (i, 0))] * 2,
    out_shape=[jax.ShapeDtypeStruct((NPAD, D), jnp.float32)] * 2,
)


def _head_body(q0, q1, d0, d1, h1r, w1, b1, w2, b2, wc, bc, out):
    deg = d0[:, 0:1] + d1[:, 0:1]
    invd = 1.0 / jnp.maximum(deg, 1.0)
    h2 = jnp.maximum((q0[...] + q1[...]) * invd + h1r[...], 0.0)
    h3 = jax.nn.gelu(jnp.dot(h2, w1[...], preferred_element_type=jnp.float32) + b1[...])
    h4 = jax.nn.gelu(jnp.dot(h3, w2[...], preferred_element_type=jnp.float32) + b2[...])
    out[...] = jnp.dot(h4, wc[...], preferred_element_type=jnp.float32) + bc[...]


_head = pl.pallas_call(
    _head_body,
    grid=(G,),
    in_specs=[
        pl.BlockSpec((RB, D), lambda i: (i, 0)),
        pl.BlockSpec((RB, D), lambda i: (i, 0)),
        pl.BlockSpec((RB, D), lambda i: (i, 0)),
        pl.BlockSpec((RB, D), lambda i: (i, 0)),
        pl.BlockSpec((RB, D), lambda i: (i, 0)),
        pl.BlockSpec((D, HIDQ), lambda i: (0, 0)),
        pl.BlockSpec((1, HIDQ), lambda i: (0, 0)),
        pl.BlockSpec((HIDQ, D), lambda i: (0, 0)),
        pl.BlockSpec((1, D), lambda i: (0, 0)),
        pl.BlockSpec((D, D), lambda i: (0, 0)),
        pl.BlockSpec((1, D), lambda i: (0, 0)),
    ],
    out_specs=pl.BlockSpec((RB, D), lambda i: (i, 0)),
    out_shape=jax.ShapeDtypeStruct((NPAD, D), jnp.float32),
)


def kernel(x, edge_index, conv0_Wl, conv0_bl, conv0_Wr,
           conv1_Wl, conv1_bl, conv1_Wr,
           kan_W1, kan_b1, kan_W2, kan_b2, cls_W, cls_b):
    src = edge_index[0].astype(jnp.int32)
    dst = edge_index[1].astype(jnp.int32)
    pad = EPAD - E
    pad_src = jnp.arange(pad, dtype=jnp.int32) % N
    pad_dst = N + jnp.arange(pad, dtype=jnp.int32) % (NPAD - N)
    srcw = jnp.concatenate([src, pad_src]).reshape(NW, K, B)
    dstw = jnp.concatenate([dst, pad_dst]).reshape(NW, K, B)
    xpad = jnp.zeros((NPAD, D), jnp.float32).at[:N].set(x)

    xl, xr = _dense0(xpad, conv0_Wl, conv0_Wr, conv0_bl.reshape(1, D))
    agg0 = _sc_agg(xl, srcw, dstw)
    degp = _sc_deg(dstw)
    p0, p1 = agg0[0], agg0[1]
    d0, d1 = degp[0], degp[1]
    h1l, h1r = _dense1(p0, p1, d0, d1, xr,
                       conv1_Wl, conv1_Wr, conv1_bl.reshape(1, D))
    agg1 = _sc_agg(h1l, srcw, dstw)
    wc = jnp.zeros((D, D), jnp.float32).at[:, 0:1].set(cls_W)
    bc = jnp.broadcast_to(cls_b.reshape(1, 1), (1, D))
    out = _head(agg1[0], agg1[1], d0, d1, h1r,
                kan_W1, kan_b1.reshape(1, HIDQ), kan_W2, kan_b2.reshape(1, D),
                wc, bc)
    return out[:N, 0]


# speedup vs baseline: 9.5128x; 1.0427x over previous
import jax
import jax.numpy as jnp
from jax import lax
from jax.experimental import pallas as pl
from jax.experimental.pallas import tpu as pltpu
from jax.experimental.pallas import tpu_sc as plsc

N = 10000
D = 128
HIDQ = 512
NPAD = 10240
TRASH = N
E = 320000
NW = 32
B = 128
K = 80
EPAD = NW * B * K
RPT = NPAD // 16
DEGW = 16
W0 = D + DEGW
RB = 1280
G = NPAD // RB


def _set2d(ref, nrows, ncols, val):
    v = jnp.full((16,), val, jnp.float32)

    def body(i, carry):
        for q in range(ncols // 16):
            ref[i, pl.ds(q * 16, 16)] = v
        return carry

    lax.fori_loop(0, nrows, body, None)


_SC_MESH = plsc.VectorSubcoreMesh(core_axis_name="c", subcore_axis_name="s")


def _sc_agg_body(table, srcw, dstw, out,
                 sidx, didx, rows0, rows1, acc,
                 isems, gsem, ssems):
    c = lax.axis_index("c")
    s = lax.axis_index("s")
    wid = c * 16 + s
    rows = (rows0, rows1)
    _set2d(rows0, B, D, 0.0)
    for k in range(RPT // B):
        pltpu.sync_copy(rows0, acc.at[pl.ds(s * RPT + k * B, B)])
    plsc.subcore_barrier()

    for u in range(2):
        pltpu.async_copy(srcw.at[wid, u], sidx.at[u], isems.at[u])
        pltpu.async_copy(dstw.at[wid, u], didx.at[u], isems.at[u])

    def outer(g, carry):
        for u in range(4):
            j = g * 4 + u
            b = u % 2
            pltpu.make_async_copy(srcw.at[wid, j], sidx.at[u], isems.at[u]).wait()
            pltpu.make_async_copy(dstw.at[wid, j], didx.at[u], isems.at[u]).wait()
            @pl.when(j >= 2)
            def _():
                pltpu.make_async_copy(table.at[pl.ds(0, B)], rows[b],
                                      ssems.at[b]).wait()
            pltpu.async_copy(table.at[sidx.at[u]], rows[b], gsem).wait()
            pltpu.async_copy(rows[b], acc.at[didx.at[u]], ssems.at[b], add=True)
            @pl.when(j + 2 < K)
            def _():
                u2 = (u + 2) % 4
                pltpu.async_copy(srcw.at[wid, j + 2], sidx.at[u2], isems.at[u2])
                pltpu.async_copy(dstw.at[wid, j + 2], didx.at[u2], isems.at[u2])
        return carry

    lax.fori_loop(0, K // 4, outer, None)
    for b in range(2):
        pltpu.make_async_copy(table.at[pl.ds(0, B)], rows[b], ssems.at[b]).wait()
    plsc.subcore_barrier()
    r0 = s * RPT
    pltpu.sync_copy(acc.at[pl.ds(r0, RPT)], out.at[c, pl.ds(r0, RPT)])


_sc_agg = pl.kernel(
    _sc_agg_body,
    out_type=jax.ShapeDtypeStruct((2, NPAD, D), jnp.float32),
    mesh=_SC_MESH,
    scratch_types=[
        pltpu.VMEM((4, B), jnp.int32),
        pltpu.VMEM((4, B), jnp.int32),
        pltpu.VMEM((B, D), jnp.float32),
        pltpu.VMEM((B, D), jnp.float32),
        pltpu.VMEM_SHARED((NPAD, D), jnp.float32),
        pltpu.SemaphoreType.DMA((4,)),
        pltpu.SemaphoreType.DMA,
        pltpu.SemaphoreType.DMA((2,)),
    ],
)


def _sc_deg_body(dstw, out, didx, ones, dacc, isems, ssems):
    c = lax.axis_index("c")
    s = lax.axis_index("s")
    wid = c * 16 + s
    _set2d(ones, B, D, 0.0)
    for k in range(RPT // B):
        pltpu.sync_copy(ones, dacc.at[pl.ds(s * RPT + k * B, B)])
    _set2d(ones, B, D, 1.0)
    plsc.subcore_barrier()

    for u in range(2):
        pltpu.async_copy(dstw.at[wid, u], didx.at[u], isems.at[u])

    def outer(g, carry):
        for u in range(4):
            j = g * 4 + u
            b = u % 2
            pltpu.make_async_copy(dstw.at[wid, j], didx.at[u], isems.at[u]).wait()
            @pl.when(j >= 2)
            def _():
                pltpu.make_async_copy(out.at[0, pl.ds(0, B)], ones,
                                      ssems.at[b]).wait()
            pltpu.async_copy(ones, dacc.at[didx.at[u]], ssems.at[b], add=True)
            @pl.when(j + 2 < K)
            def _():
                u2 = (u + 2) % 4
                pltpu.async_copy(dstw.at[wid, j + 2], didx.at[u2], isems.at[u2])
        return carry

    lax.fori_loop(0, K // 4, outer, None)
    for b in range(2):
        pltpu.make_async_copy(out.at[0, pl.ds(0, B)], ones, ssems.at[b]).wait()
    plsc.subcore_barrier()
    r0 = s * RPT
    pltpu.sync_copy(dacc.at[pl.ds(r0, RPT)], out.at[c, pl.ds(r0, RPT)])


_sc_deg = pl.kernel(
    _sc_deg_body,
    out_type=jax.ShapeDtypeStruct((2, NPAD, D), jnp.float32),
    mesh=_SC_MESH,
    scratch_types=[
        pltpu.VMEM((4, B), jnp.int32),
        pltpu.VMEM((B, D), jnp.float32),
        pltpu.VMEM_SHARED((NPAD, D), jnp.float32),
        pltpu.SemaphoreType.DMA((4,)),
        pltpu.SemaphoreType.DMA((2,)),
    ],
)


def _dense0_body(x_ref, wl, wr, bl, xl_o, xr_o):
    xv = x_ref[...]
    xl_o[...] = jnp.dot(xv, wl[...], preferred_element_type=jnp.float32)
    xr_o[...] = jnp.dot(xv, wr[...], preferred_element_type=jnp.float32) + bl[...]


_dense0 = pl.pallas_call(
    _dense0_body,
    grid=(G,),
    in_specs=[
        pl.BlockSpec((RB, D), lambda i: (i, 0)),
        pl.BlockSpec((D, D), lambda i: (0, 0)),
        pl.BlockSpec((D, D), lambda i: (0, 0)),
        pl.BlockSpec((1, D), lambda i: (0, 0)),
    ],
    out_specs=[pl.BlockSpec((RB, D), lambda i: (i, 0))] * 2,
    out_shape=[jax.ShapeDtypeStruct((N, D), jnp.float32)] * 2,
)


def _dense1_body(p, d, xr, wl, wr, bl, h1l_o, h1r_o):
    deg = d[0, :, 0:1] + d[1, :, 0:1]
    invd = 1.0 / jnp.maximum(deg, 1.0)
    h1 = jnp.maximum((p[0] + p[1]) * invd + xr[...], 0.0)
    h1l_o[...] = jnp.dot(h1, wl[...], preferred_element_type=jnp.float32)
    h1r_o[...] = jnp.dot(h1, wr[...], preferred_element_type=jnp.float32) + bl[...]


_dense1 = pl.pallas_call(
    _dense1_body,
    grid=(G,),
    in_specs=[
        pl.BlockSpec((2, RB, D), lambda i: (0, i, 0)),
        pl.BlockSpec((2, RB, D), lambda i: (0, i, 0)),
        pl.BlockSpec((RB, D), lambda i: (i, 0)),
        pl.BlockSpec((D, D), lambda i: (0, 0)),
        pl.BlockSpec((D, D), lambda i: (0, 0)),
        pl.BlockSpec((1, D), lambda i: (0, 0)),
    ],
    out_specs=[pl.BlockSpec((RB, D), lambda i: (i, 0))] * 2,
    out_shape=[jax.ShapeDtypeStruct((N, D), jnp.float32)] * 2,
)


def _head_body(q, d, h1r, w1, b1, w2, b2, wc, bc, out):
    deg = d[0, :, 0:1] + d[1, :, 0:1]
    invd = 1.0 / jnp.maximum(deg, 1.0)
    h2 = jnp.maximum((q[0] + q[1]) * invd + h1r[...], 0.0)
    h3 = jax.nn.gelu(jnp.dot(h2, w1[...], preferred_element_type=jnp.float32) + b1[...])
    h4 = jax.nn.gelu(jnp.dot(h3, w2[...], preferred_element_type=jnp.float32) + b2[...])
    out[...] = jnp.dot(h4, wc[...], preferred_element_type=jnp.float32) + bc[...]


_head = pl.pallas_call(
    _head_body,
    grid=(G,),
    in_specs=[
        pl.BlockSpec((2, RB, D), lambda i: (0, i, 0)),
        pl.BlockSpec((2, RB, D), lambda i: (0, i, 0)),
        pl.BlockSpec((RB, D), lambda i: (i, 0)),
        pl.BlockSpec((D, HIDQ), lambda i: (0, 0)),
        pl.BlockSpec((1, HIDQ), lambda i: (0, 0)),
        pl.BlockSpec((HIDQ, D), lambda i: (0, 0)),
        pl.BlockSpec((1, D), lambda i: (0, 0)),
        pl.BlockSpec((D, D), lambda i: (0, 0)),
        pl.BlockSpec((1, D), lambda i: (0, 0)),
    ],
    out_specs=pl.BlockSpec((RB, D), lambda i: (i, 0)),
    out_shape=jax.ShapeDtypeStruct((N, D), jnp.float32),
)


def kernel(x, edge_index, conv0_Wl, conv0_bl, conv0_Wr,
           conv1_Wl, conv1_bl, conv1_Wr,
           kan_W1, kan_b1, kan_W2, kan_b2, cls_W, cls_b):
    src = edge_index[0].astype(jnp.int32)
    dst = edge_index[1].astype(jnp.int32)
    pad = EPAD - E
    pad_src = jnp.arange(pad, dtype=jnp.int32) % N
    pad_dst = N + jnp.arange(pad, dtype=jnp.int32) % (NPAD - N)
    srcw = jnp.concatenate([src, pad_src]).reshape(NW, K, B)
    dstw = jnp.concatenate([dst, pad_dst]).reshape(NW, K, B)

    xl, xr = _dense0(x, conv0_Wl, conv0_Wr, conv0_bl.reshape(1, D))
    agg0 = _sc_agg(xl, srcw, dstw)
    degp = _sc_deg(dstw)
    h1l, h1r = _dense1(agg0, degp, xr,
                       conv1_Wl, conv1_Wr, conv1_bl.reshape(1, D))
    agg1 = _sc_agg(h1l, srcw, dstw)
    wc = jnp.zeros((D, D), jnp.float32).at[:, 0:1].set(cls_W)
    bc = jnp.broadcast_to(cls_b.reshape(1, 1), (1, D))
    out = _head(agg1, degp, h1r,
                kan_W1, kan_b1.reshape(1, HIDQ), kan_W2, kan_b2.reshape(1, D),
                wc, bc)
    return out[:, 0]


# speedup vs baseline: 11.2583x; 1.1835x over previous
import jax
import jax.numpy as jnp
from jax import lax
from jax.experimental import pallas as pl
from jax.experimental.pallas import tpu as pltpu
from jax.experimental.pallas import tpu_sc as plsc

N = 10000
D = 128
HIDQ = 512
NPAD = 10240
TRASH = N
E = 320000
NW = 32
B = 128
K = 80
EPAD = NW * B * K
RPT = NPAD // 16
DEGW = 16
W0 = D + DEGW
RB = 1280
G = NPAD // RB


def _set2d(ref, nrows, ncols, val):
    v = jnp.full((16,), val, jnp.float32)

    def body(i, carry):
        for q in range(ncols // 16):
            ref[i, pl.ds(q * 16, 16)] = v
        return carry

    lax.fori_loop(0, nrows, body, None)


_SC_MESH = plsc.VectorSubcoreMesh(core_axis_name="c", subcore_axis_name="s")


def _sc_agg_body(table, srcw, dstw, out,
                 sidx, didx, rows0, rows1, acc,
                 isems, gsem, ssems):
    c = lax.axis_index("c")
    s = lax.axis_index("s")
    wid = c * 16 + s
    rows = (rows0, rows1)
    _set2d(rows0, B, D, 0.0)
    for k in range(RPT // B):
        pltpu.sync_copy(rows0, acc.at[pl.ds(s * RPT + k * B, B)])
    plsc.subcore_barrier()

    for u in range(2):
        pltpu.async_copy(srcw.at[wid, u], sidx.at[u], isems.at[u])
        pltpu.async_copy(dstw.at[wid, u], didx.at[u], isems.at[u])

    def outer(g, carry):
        for u in range(4):
            j = g * 4 + u
            b = u % 2
            pltpu.make_async_copy(srcw.at[wid, j], sidx.at[u], isems.at[u]).wait()
            pltpu.make_async_copy(dstw.at[wid, j], didx.at[u], isems.at[u]).wait()
            @pl.when(j >= 2)
            def _():
                pltpu.make_async_copy(table.at[pl.ds(0, B)], rows[b],
                                      ssems.at[b]).wait()
            pltpu.async_copy(table.at[sidx.at[u]], rows[b], gsem).wait()
            pltpu.async_copy(rows[b], acc.at[didx.at[u]], ssems.at[b], add=True)
            @pl.when(j + 2 < K)
            def _():
                u2 = (u + 2) % 4
                pltpu.async_copy(srcw.at[wid, j + 2], sidx.at[u2], isems.at[u2])
                pltpu.async_copy(dstw.at[wid, j + 2], didx.at[u2], isems.at[u2])
        return carry

    lax.fori_loop(0, K // 4, outer, None)
    for b in range(2):
        pltpu.make_async_copy(table.at[pl.ds(0, B)], rows[b], ssems.at[b]).wait()
    plsc.subcore_barrier()
    r0 = s * RPT
    pltpu.sync_copy(acc.at[pl.ds(r0, RPT)], out.at[c, pl.ds(r0, RPT)])


_sc_agg = pl.kernel(
    _sc_agg_body,
    out_type=jax.ShapeDtypeStruct((2, NPAD, D), jnp.float32),
    mesh=_SC_MESH,
    scratch_types=[
        pltpu.VMEM((4, B), jnp.int32),
        pltpu.VMEM((4, B), jnp.int32),
        pltpu.VMEM((B, D), jnp.float32),
        pltpu.VMEM((B, D), jnp.float32),
        pltpu.VMEM_SHARED((NPAD, D), jnp.float32),
        pltpu.SemaphoreType.DMA((4,)),
        pltpu.SemaphoreType.DMA,
        pltpu.SemaphoreType.DMA((2,)),
    ],
)


DROWS = NPAD // D


def _sc_deg_body(dstw, out, didx_all, dloc, ridx, dacc, sem):
    c = lax.axis_index("c")
    s = lax.axis_index("s")
    wid = c * 16 + s
    pltpu.sync_copy(dstw.at[wid], didx_all)
    _set2d(dloc, DROWS, D, 0.0)
    iota = lax.iota(jnp.int32, 16)
    for k in range(DROWS // 16):
        ridx[pl.ds(k * 16, 16)] = iota + (k * 16)
    @pl.when(s == 0)
    def _():
        pltpu.sync_copy(dloc, dacc)
    plsc.subcore_barrier()

    def step(j, carry):
        for q in range(B // 16):
            d16 = didx_all[j, pl.ds(q * 16, 16)]
            cnt, last = plsc.scan_count(d16)
            plsc.addupdate_scatter(
                dloc, [lax.shift_right_logical(d16, 7), lax.bitwise_and(d16, 127)],
                cnt.astype(jnp.float32), mask=last)
        return carry

    lax.fori_loop(0, K, step, None)
    pltpu.async_copy(dloc, dacc.at[ridx], sem, add=True).wait()
    plsc.subcore_barrier()
    @pl.when(s < DROWS // 8)
    def _():
        r0 = s * 8
        pltpu.sync_copy(dacc.at[pl.ds(r0, 8)], out.at[c, pl.ds(r0, 8)])


_sc_deg = pl.kernel(
    _sc_deg_body,
    out_type=jax.ShapeDtypeStruct((2, DROWS, D), jnp.float32),
    mesh=_SC_MESH,
    compiler_params=pltpu.CompilerParams(needs_layout_passes=False),
    scratch_types=[
        pltpu.VMEM((K, B), jnp.int32),
        pltpu.VMEM((DROWS, D), jnp.float32),
        pltpu.VMEM((DROWS,), jnp.int32),
        pltpu.VMEM_SHARED((DROWS, D), jnp.float32),
        pltpu.SemaphoreType.DMA,
    ],
)


def _dense0_body(x_ref, wl, wr, bl, xl_o, xr_o):
    xv = x_ref[...]
    xl_o[...] = jnp.dot(xv, wl[...], preferred_element_type=jnp.float32)
    xr_o[...] = jnp.dot(xv, wr[...], preferred_element_type=jnp.float32) + bl[...]


_dense0 = pl.pallas_call(
    _dense0_body,
    grid=(G,),
    in_specs=[
        pl.BlockSpec((RB, D), lambda i: (i, 0)),
        pl.BlockSpec((D, D), lambda i: (0, 0)),
        pl.BlockSpec((D, D), lambda i: (0, 0)),
        pl.BlockSpec((1, D), lambda i: (0, 0)),
    ],
    out_specs=[pl.BlockSpec((RB, D), lambda i: (i, 0))] * 2,
    out_shape=[jax.ShapeDtypeStruct((N, D), jnp.float32)] * 2,
)


def _dense1_body(p, d, xr, wl, wr, bl, h1l_o, h1r_o):
    deg = d[0] + d[1]
    invd = 1.0 / jnp.maximum(deg, 1.0)
    h1 = jnp.maximum((p[0] + p[1]) * invd + xr[...], 0.0)
    h1l_o[...] = jnp.dot(h1, wl[...], preferred_element_type=jnp.float32)
    h1r_o[...] = jnp.dot(h1, wr[...], preferred_element_type=jnp.float32) + bl[...]


_dense1 = pl.pallas_call(
    _dense1_body,
    grid=(G,),
    in_specs=[
        pl.BlockSpec((2, RB, D), lambda i: (0, i, 0)),
        pl.BlockSpec((2, RB, 1), lambda i: (0, i, 0)),
        pl.BlockSpec((RB, D), lambda i: (i, 0)),
        pl.BlockSpec((D, D), lambda i: (0, 0)),
        pl.BlockSpec((D, D), lambda i: (0, 0)),
        pl.BlockSpec((1, D), lambda i: (0, 0)),
    ],
    out_specs=[pl.BlockSpec((RB, D), lambda i: (i, 0))] * 2,
    out_shape=[jax.ShapeDtypeStruct((N, D), jnp.float32)] * 2,
)


def _head_body(q, d, h1r, w1, b1, w2, b2, wc, bc, out):
    deg = d[0] + d[1]
    invd = 1.0 / jnp.maximum(deg, 1.0)
    h2 = jnp.maximum((q[0] + q[1]) * invd + h1r[...], 0.0)
    h3 = jax.nn.gelu(jnp.dot(h2, w1[...], preferred_element_type=jnp.float32) + b1[...])
    h4 = jax.nn.gelu(jnp.dot(h3, w2[...], preferred_element_type=jnp.float32) + b2[...])
    out[...] = jnp.dot(h4, wc[...], preferred_element_type=jnp.float32) + bc[...]


_head = pl.pallas_call(
    _head_body,
    grid=(G,),
    in_specs=[
        pl.BlockSpec((2, RB, D), lambda i: (0, i, 0)),
        pl.BlockSpec((2, RB, 1), lambda i: (0, i, 0)),
        pl.BlockSpec((RB, D), lambda i: (i, 0)),
        pl.BlockSpec((D, HIDQ), lambda i: (0, 0)),
        pl.BlockSpec((1, HIDQ), lambda i: (0, 0)),
        pl.BlockSpec((HIDQ, D), lambda i: (0, 0)),
        pl.BlockSpec((1, D), lambda i: (0, 0)),
        pl.BlockSpec((D, D), lambda i: (0, 0)),
        pl.BlockSpec((1, D), lambda i: (0, 0)),
    ],
    out_specs=pl.BlockSpec((RB, D), lambda i: (i, 0)),
    out_shape=jax.ShapeDtypeStruct((N, D), jnp.float32),
)


def kernel(x, edge_index, conv0_Wl, conv0_bl, conv0_Wr,
           conv1_Wl, conv1_bl, conv1_Wr,
           kan_W1, kan_b1, kan_W2, kan_b2, cls_W, cls_b):
    src = edge_index[0].astype(jnp.int32)
    dst = edge_index[1].astype(jnp.int32)
    pad = EPAD - E
    pad_src = jnp.arange(pad, dtype=jnp.int32) % N
    pad_dst = N + jnp.arange(pad, dtype=jnp.int32) % (NPAD - N)
    srcw = jnp.concatenate([src, pad_src]).reshape(NW, K, B)
    dstw = jnp.concatenate([dst, pad_dst]).reshape(NW, K, B)

    xl, xr = _dense0(x, conv0_Wl, conv0_Wr, conv0_bl.reshape(1, D))
    agg0 = _sc_agg(xl, srcw, dstw)
    degp = _sc_deg(dstw).reshape(2, NPAD, 1)
    h1l, h1r = _dense1(agg0, degp, xr,
                       conv1_Wl, conv1_Wr, conv1_bl.reshape(1, D))
    agg1 = _sc_agg(h1l, srcw, dstw)
    wc = jnp.zeros((D, D), jnp.float32).at[:, 0:1].set(cls_W)
    bc = jnp.broadcast_to(cls_b.reshape(1, 1), (1, D))
    out = _head(agg1, degp, h1r,
                kan_W1, kan_b1.reshape(1, HIDQ), kan_W2, kan_b2.reshape(1, D),
                wc, bc)
    return out[:, 0]


# speedup vs baseline: 13.6147x; 1.2093x over previous
import jax
import jax.numpy as jnp
from jax import lax
from jax.experimental import pallas as pl
from jax.experimental.pallas import tpu as pltpu
from jax.experimental.pallas import tpu_sc as plsc

N = 10000
D = 128
HIDQ = 512
NPAD = 10240
TRASH = N
E = 320000
NW = 32
B = 96
K = 108
EPAD = NW * B * K
RPT = NPAD // 16
DEGW = 16
W0 = D + DEGW
RB = 1280
G = NPAD // RB


def _set2d(ref, nrows, ncols, val):
    v = jnp.full((16,), val, jnp.float32)

    def body(i, carry):
        for q in range(ncols // 16):
            ref[i, pl.ds(q * 16, 16)] = v
        return carry

    lax.fori_loop(0, nrows, body, None)


_SC_MESH = plsc.VectorSubcoreMesh(core_axis_name="c", subcore_axis_name="s")


def _sc_agg_body(table, srcw, dstw, out,
                 sidx, didx, rows0, rows1, rows2, acc,
                 isems, gsems, ssems):
    c = lax.axis_index("c")
    s = lax.axis_index("s")
    wid = c * 16 + s
    rows = (rows0, rows1, rows2)
    _set2d(rows0, B, D, 0.0)
    for k in range(RPT // B):
        pltpu.sync_copy(rows0, acc.at[pl.ds(s * RPT + k * B, B)])
    rem = RPT % B
    if rem:
        pltpu.sync_copy(rows0.at[pl.ds(0, rem)],
                        acc.at[pl.ds(s * RPT + (RPT // B) * B, rem)])
    plsc.subcore_barrier()

    for u in range(3):
        pltpu.async_copy(srcw.at[wid, u], sidx.at[u], isems.at[u])
        pltpu.async_copy(dstw.at[wid, u], didx.at[u], isems.at[u])

    def outer(g, carry):
        for u in range(6):
            j = g * 6 + u
            b = u % 3
            p = u % 2
            b1 = (u + 2) % 3
            p1 = (u + 1) % 2
            u1 = (u + 5) % 6
            pltpu.make_async_copy(srcw.at[wid, j], sidx.at[u], isems.at[u]).wait()
            pltpu.make_async_copy(dstw.at[wid, j], didx.at[u], isems.at[u]).wait()
            @pl.when(j >= 3)
            def _():
                pltpu.make_async_copy(table.at[pl.ds(0, B)], rows[b],
                                      ssems.at[b]).wait()
            pltpu.async_copy(table.at[sidx.at[u]], rows[b], gsems.at[p])
            @pl.when(j >= 1)
            def _():
                pltpu.make_async_copy(table.at[pl.ds(0, B)], rows[b1],
                                      gsems.at[p1]).wait()
                pltpu.async_copy(rows[b1], acc.at[didx.at[u1]],
                                 ssems.at[b1], add=True)
            @pl.when(j + 3 < K)
            def _():
                u3 = (u + 3) % 6
                pltpu.async_copy(srcw.at[wid, j + 3], sidx.at[u3], isems.at[u3])
                pltpu.async_copy(dstw.at[wid, j + 3], didx.at[u3], isems.at[u3])
        return carry

    lax.fori_loop(0, K // 6, outer, None)
    bl = (K - 1) % 3
    pltpu.make_async_copy(table.at[pl.ds(0, B)], rows[bl],
                          gsems.at[(K - 1) % 2]).wait()
    pltpu.async_copy(rows[bl], acc.at[didx.at[(K - 1) % 6]],
                     ssems.at[bl], add=True)
    for b in range(3):
        pltpu.make_async_copy(table.at[pl.ds(0, B)], rows[b], ssems.at[b]).wait()
    plsc.subcore_barrier()
    r0 = s * RPT
    pltpu.sync_copy(acc.at[pl.ds(r0, RPT)], out.at[c, pl.ds(r0, RPT)])


_sc_agg = pl.kernel(
    _sc_agg_body,
    out_type=jax.ShapeDtypeStruct((2, NPAD, D), jnp.float32),
    mesh=_SC_MESH,
    scratch_types=[
        pltpu.VMEM((6, B), jnp.int32),
        pltpu.VMEM((6, B), jnp.int32),
        pltpu.VMEM((B, D), jnp.float32),
        pltpu.VMEM((B, D), jnp.float32),
        pltpu.VMEM((B, D), jnp.float32),
        pltpu.VMEM_SHARED((NPAD, D), jnp.float32),
        pltpu.SemaphoreType.DMA((6,)),
        pltpu.SemaphoreType.DMA((2,)),
        pltpu.SemaphoreType.DMA((3,)),
    ],
)


DROWS = NPAD // D


def _sc_deg_body(dstw, out, didx_all, dloc, ridx, dacc, sem):
    c = lax.axis_index("c")
    s = lax.axis_index("s")
    wid = c * 16 + s
    pltpu.sync_copy(dstw.at[wid], didx_all)
    _set2d(dloc, DROWS, D, 0.0)
    iota = lax.iota(jnp.int32, 16)
    for k in range(DROWS // 16):
        ridx[pl.ds(k * 16, 16)] = iota + (k * 16)
    @pl.when(s == 0)
    def _():
        pltpu.sync_copy(dloc, dacc)
    plsc.subcore_barrier()

    def step(j, carry):
        for q in range(B // 16):
            d16 = didx_all[j, pl.ds(q * 16, 16)]
            cnt, last = plsc.scan_count(d16)
            plsc.addupdate_scatter(
                dloc, [lax.shift_right_logical(d16, 7), lax.bitwise_and(d16, 127)],
                cnt.astype(jnp.float32), mask=last)
        return carry

    lax.fori_loop(0, K, step, None)
    pltpu.async_copy(dloc, dacc.at[ridx], sem, add=True).wait()
    plsc.subcore_barrier()
    @pl.when(s < DROWS // 8)
    def _():
        r0 = s * 8
        pltpu.sync_copy(dacc.at[pl.ds(r0, 8)], out.at[c, pl.ds(r0, 8)])


_sc_deg = pl.kernel(
    _sc_deg_body,
    out_type=jax.ShapeDtypeStruct((2, DROWS, D), jnp.float32),
    mesh=_SC_MESH,
    compiler_params=pltpu.CompilerParams(needs_layout_passes=False),
    scratch_types=[
        pltpu.VMEM((K, B), jnp.int32),
        pltpu.VMEM((DROWS, D), jnp.float32),
        pltpu.VMEM((DROWS,), jnp.int32),
        pltpu.VMEM_SHARED((DROWS, D), jnp.float32),
        pltpu.SemaphoreType.DMA,
    ],
)


def _dense0_body(x_ref, wl, wr, bl, xl_o, xr_o):
    xv = x_ref[...]
    xl_o[...] = jnp.dot(xv, wl[...], preferred_element_type=jnp.float32)
    xr_o[...] = jnp.dot(xv, wr[...], preferred_element_type=jnp.float32) + bl[...]


_dense0 = pl.pallas_call(
    _dense0_body,
    grid=(G,),
    in_specs=[
        pl.BlockSpec((RB, D), lambda i: (i, 0)),
        pl.BlockSpec((D, D), lambda i: (0, 0)),
        pl.BlockSpec((D, D), lambda i: (0, 0)),
        pl.BlockSpec((1, D), lambda i: (0, 0)),
    ],
    out_specs=[pl.BlockSpec((RB, D), lambda i: (i, 0))] * 2,
    out_shape=[jax.ShapeDtypeStruct((N, D), jnp.float32)] * 2,
)


def _dense1_body(p, d, xr, wl, wr, bl, h1l_o, h1r_o):
    deg = d[0] + d[1]
    invd = 1.0 / jnp.maximum(deg, 1.0)
    h1 = jnp.maximum((p[0] + p[1]) * invd + xr[...], 0.0)
    h1l_o[...] = jnp.dot(h1, wl[...], preferred_element_type=jnp.float32)
    h1r_o[...] = jnp.dot(h1, wr[...], preferred_element_type=jnp.float32) + bl[...]


_dense1 = pl.pallas_call(
    _dense1_body,
    grid=(G,),
    in_specs=[
        pl.BlockSpec((2, RB, D), lambda i: (0, i, 0)),
        pl.BlockSpec((2, RB, 1), lambda i: (0, i, 0)),
        pl.BlockSpec((RB, D), lambda i: (i, 0)),
        pl.BlockSpec((D, D), lambda i: (0, 0)),
        pl.BlockSpec((D, D), lambda i: (0, 0)),
        pl.BlockSpec((1, D), lambda i: (0, 0)),
    ],
    out_specs=[pl.BlockSpec((RB, D), lambda i: (i, 0))] * 2,
    out_shape=[jax.ShapeDtypeStruct((N, D), jnp.float32)] * 2,
)


def _head_body(q, d, h1r, w1, b1, w2, b2, wc, bc, out):
    deg = d[0] + d[1]
    invd = 1.0 / jnp.maximum(deg, 1.0)
    h2 = jnp.maximum((q[0] + q[1]) * invd + h1r[...], 0.0)
    h3 = jax.nn.gelu(jnp.dot(h2, w1[...], preferred_element_type=jnp.float32) + b1[...])
    h4 = jax.nn.gelu(jnp.dot(h3, w2[...], preferred_element_type=jnp.float32) + b2[...])
    out[...] = jnp.dot(h4, wc[...], preferred_element_type=jnp.float32) + bc[...]


_head = pl.pallas_call(
    _head_body,
    grid=(G,),
    in_specs=[
        pl.BlockSpec((2, RB, D), lambda i: (0, i, 0)),
        pl.BlockSpec((2, RB, 1), lambda i: (0, i, 0)),
        pl.BlockSpec((RB, D), lambda i: (i, 0)),
        pl.BlockSpec((D, HIDQ), lambda i: (0, 0)),
        pl.BlockSpec((1, HIDQ), lambda i: (0, 0)),
        pl.BlockSpec((HIDQ, D), lambda i: (0, 0)),
        pl.BlockSpec((1, D), lambda i: (0, 0)),
        pl.BlockSpec((D, D), lambda i: (0, 0)),
        pl.BlockSpec((1, D), lambda i: (0, 0)),
    ],
    out_specs=pl.BlockSpec((RB, D), lambda i: (i, 0)),
    out_shape=jax.ShapeDtypeStruct((N, D), jnp.float32),
)


def kernel(x, edge_index, conv0_Wl, conv0_bl, conv0_Wr,
           conv1_Wl, conv1_bl, conv1_Wr,
           kan_W1, kan_b1, kan_W2, kan_b2, cls_W, cls_b):
    src = edge_index[0].astype(jnp.int32)
    dst = edge_index[1].astype(jnp.int32)
    pad = EPAD - E
    pad_src = jnp.arange(pad, dtype=jnp.int32) % N
    pad_dst = N + jnp.arange(pad, dtype=jnp.int32) % (NPAD - N)
    srcw = jnp.concatenate([src, pad_src]).reshape(NW, K, B)
    dstw = jnp.concatenate([dst, pad_dst]).reshape(NW, K, B)

    xl, xr = _dense0(x, conv0_Wl, conv0_Wr, conv0_bl.reshape(1, D))
    agg0 = _sc_agg(xl, srcw, dstw)
    degp = _sc_deg(dstw).reshape(2, NPAD, 1)
    h1l, h1r = _dense1(agg0, degp, xr,
                       conv1_Wl, conv1_Wr, conv1_bl.reshape(1, D))
    agg1 = _sc_agg(h1l, srcw, dstw)
    wc = jnp.zeros((D, D), jnp.float32).at[:, 0:1].set(cls_W)
    bc = jnp.broadcast_to(cls_b.reshape(1, 1), (1, D))
    out = _head(agg1, degp, h1r,
                kan_W1, kan_b1.reshape(1, HIDQ), kan_W2, kan_b2.reshape(1, D),
                wc, bc)
    return out[:, 0]
